# Initial kernel scaffold; baseline (speedup 1.0000x reference)
import jax
import jax.numpy as jnp
from jax.experimental import pallas as pl


def kernel(x, y, source_domain_indicator, node_map_edge_index, intersect_indicator, domain_map_edge_index, W_xsum, W_xint, W_y, bn_weight, bn_bias):
    raise NotImplementedError("write your pallas kernel here")


# speedup vs baseline: 5.8859x; 5.8859x over previous
import functools

import jax
import jax.numpy as jnp
from jax import lax
from jax.experimental import pallas as pl
from jax.experimental.pallas import tpu as pltpu
from jax.experimental.pallas import tpu_sc as plsc

H = 128
NX = 10000
NY = 5000
NSRC = 5000
EN = 640000
ED = 64000

NC = 2
NS = 16
NW = NC * NS

SB = 512
NSRC_PAD = 5120
NSEG = NSRC_PAD + ED
NB = NSEG // SB
ENX = EN + NX
EPAD = 256
CE = 128
ACC_ROWS = SB + 8

PR = 5120
PRT = PR // NS

_mesh = plsc.VectorSubcoreMesh(core_axis_name="c", subcore_axis_name="s")
_f32 = jnp.float32
_i32 = jnp.int32


def _zero_vmem_2d(ref, rows):
    def body(i, _):
        r = i // (H // 16)
        g = i % (H // 16)
        ref[r, pl.ds(g * 16, 16)] = jnp.zeros((16,), _f32)
        return 0
    lax.fori_loop(0, rows * (H // 16), body, 0, unroll=False)


def _k1_body(x_h, src_h, loc_h, bnd_h, out_h,
             bnd_v, sbuf, lbuf, rowbuf, zbuf, acc_sh, gsem, ssem):
    c = lax.axis_index("c")
    s = lax.axis_index("s")
    wid = s * NC + c
    pltpu.sync_copy(bnd_h, bnd_v)
    _zero_vmem_2d(zbuf, 104)
    iota = lax.broadcasted_iota(_i32, (16,), 0)
    tile_base = pl.multiple_of(s * ACC_ROWS, 8)

    for bi in range(-(-NB // NW)):
        b = wid + NW * bi

        @pl.when(b < NB)
        def _():
            bv = bnd_v[pl.ds(b, 16)]
            lo = bv[0]
            hi = bv[1]
            for r in range(5):
                pltpu.sync_copy(
                    zbuf, acc_sh.at[pl.ds(tile_base + r * 104, 104)])
            a0 = pl.multiple_of(lo & (-8), 8)
            nch = (hi - a0 + CE - 1) // CE

            def chunk(i, _):
                a = pl.multiple_of(a0 + i * CE, 8)
                pltpu.sync_copy(src_h.at[pl.ds(a, CE)], sbuf)
                pltpu.sync_copy(loc_h.at[pl.ds(a, CE)], lbuf)
                for g in range(CE // 16):
                    e = a + g * 16 + iota
                    lv = lbuf[pl.ds(g * 16, 16)]
                    ok = (e >= lo) & (e < hi)
                    lbuf[pl.ds(g * 16, 16)] = tile_base + jnp.where(ok, lv, SB)
                pltpu.async_copy(x_h.at[sbuf], rowbuf, gsem).wait()
                pltpu.async_copy(rowbuf, acc_sh.at[lbuf], ssem, add=True).wait()
                return 0

            lax.fori_loop(0, nch, chunk, 0, unroll=False)
            pltpu.sync_copy(acc_sh.at[pl.ds(tile_base, SB)],
                            out_h.at[pl.ds(pl.multiple_of(b * SB, 8), SB)])


_k1 = pl.kernel(
    _k1_body,
    out_type=jax.ShapeDtypeStruct((NSEG, H), _f32),
    mesh=_mesh,
    scratch_types=[
        pltpu.VMEM((NB + 17,), _i32),
        pltpu.VMEM((CE,), _i32),
        pltpu.VMEM((CE,), _i32),
        pltpu.VMEM((CE, H), _f32),
        pltpu.VMEM((104, H), _f32),
        pltpu.VMEM_SHARED((NS * ACC_ROWS, H), _f32),
        pltpu.SemaphoreType.DMA,
        pltpu.SemaphoreType.DMA,
    ],
)


def _k2_body(xi_ref, sa_ref, y_ref, wi_ref, ws_ref, wy_ref,
             b_out, a_out, c_out):
    i = pl.program_id(0)
    b_out[...] = jnp.dot(xi_ref[...], wi_ref[...],
                         precision=lax.Precision.HIGHEST)

    @pl.when(i == 0)
    def _():
        a_out[...] = jnp.dot(sa_ref[...], ws_ref[...],
                             precision=lax.Precision.HIGHEST)
        wy = wy_ref[:, :H] + wy_ref[:, H:]
        c_out[...] = jnp.dot(y_ref[...], wy, precision=lax.Precision.HIGHEST)


def _k2(sums, y, w_xint, w_xsum, w_y):
    nblk = ED // SB
    return pl.pallas_call(
        _k2_body,
        grid=(nblk,),
        in_specs=[
            pl.BlockSpec((SB, H), lambda i: (NSRC_PAD // SB + i, 0)),
            pl.BlockSpec((NSRC_PAD, H), lambda i: (0, 0)),
            pl.BlockSpec((NY, H), lambda i: (0, 0)),
            pl.BlockSpec((H, H), lambda i: (0, 0)),
            pl.BlockSpec((H, H), lambda i: (0, 0)),
            pl.BlockSpec((H, 2 * H), lambda i: (0, 0)),
        ],
        out_specs=[
            pl.BlockSpec((SB, H), lambda i: (i, 0)),
            pl.BlockSpec((NSRC_PAD, H), lambda i: (0, 0)),
            pl.BlockSpec((NY, H), lambda i: (0, 0)),
        ],
        out_shape=[
            jax.ShapeDtypeStruct((ED, H), _f32),
            jax.ShapeDtypeStruct((NSRC_PAD, H), _f32),
            jax.ShapeDtypeStruct((NY, H), _f32),
        ],
    )(sums, sums, y, w_xint, w_xsum, w_y)


NCH3 = ED // CE


def _k3_body(b_h, a_h, c_h, dm0_h, dm1_h, m_h, part_h,
             i0, i1, abuf, cbuf, bbuf, pbuf, g0, g1):
    c = lax.axis_index("c")
    s = lax.axis_index("s")
    wid = s * NC + c
    for g in range(16):
        pbuf[pl.ds(g * 16, 16)] = jnp.zeros((16,), _f32)

    for j in range(-(-NCH3 // NW)):
        cid = wid + NW * j

        @pl.when(cid < NCH3)
        def _():
            a = pl.multiple_of(cid * CE, 8)
            pltpu.sync_copy(dm0_h.at[pl.ds(a, CE)], i0)
            pltpu.sync_copy(dm1_h.at[pl.ds(a, CE)], i1)
            pltpu.async_copy(a_h.at[i0], abuf, g0).wait()
            pltpu.async_copy(c_h.at[i1], cbuf, g1).wait()
            pltpu.sync_copy(b_h.at[pl.ds(a, CE)], bbuf)

            def row(r, carry):
                out = []
                for g in range(8):
                    col = pl.ds(g * 16, 16)
                    v = bbuf[r, col] + abuf[r, col] + cbuf[r, col]
                    bbuf[r, col] = v
                    out.append(carry[g] + v)
                    out.append(carry[8 + g] + v * v)
                return tuple(out[0::2]) + tuple(out[1::2])

            z16 = tuple(jnp.zeros((16,), _f32) for _ in range(16))
            acc = lax.fori_loop(0, CE, row, z16, unroll=False)
            for g in range(8):
                p = pbuf[pl.ds(g * 16, 16)]
                pbuf[pl.ds(g * 16, 16)] = p + acc[g]
                q = pbuf[pl.ds(128 + g * 16, 16)]
                pbuf[pl.ds(128 + g * 16, 16)] = q + acc[8 + g]
            pltpu.sync_copy(bbuf, m_h.at[pl.ds(a, CE)])

    pltpu.sync_copy(pbuf,
                    part_h.at[pl.ds(pl.multiple_of(wid * 2 * H, 8), 2 * H)])


_k3 = pl.kernel(
    _k3_body,
    out_type=(
        jax.ShapeDtypeStruct((ED, H), _f32),
        jax.ShapeDtypeStruct((NW * 2 * H,), _f32),
    ),
    mesh=_mesh,
    scratch_types=[
        pltpu.VMEM((CE,), _i32),
        pltpu.VMEM((CE,), _i32),
        pltpu.VMEM((CE, H), _f32),
        pltpu.VMEM((CE, H), _f32),
        pltpu.VMEM((CE, H), _f32),
        pltpu.VMEM((2 * H,), _f32),
        pltpu.SemaphoreType.DMA,
        pltpu.SemaphoreType.DMA,
    ],
)


def _k4_body(p_ref, w_ref, b_ref, o_ref):
    p = p_ref[...]
    tot = jnp.sum(p, axis=0, keepdims=True)
    s1 = tot[:, :H]
    s2 = tot[:, H:]
    mean = s1 / float(ED)
    var = s2 / float(ED) - mean * mean
    inv = lax.rsqrt(var + 1e-5)
    scale = w_ref[...] * inv
    shift = b_ref[...] - mean * scale
    o_ref[0:1, :] = scale
    o_ref[1:2, :] = shift


def _k4(parts, bnw, bnb):
    return pl.pallas_call(
        _k4_body,
        out_shape=jax.ShapeDtypeStruct((2, H), _f32),
    )(parts, bnw, bnb)


NCH5 = ED // CE
NCH5C = NCH5 // NC


def _k5_body(m_h, dm1_h, ss_h, pres_h,
             i1, mbuf, zbuf, ssbuf, acc_sh, sem):
    c = lax.axis_index("c")
    s = lax.axis_index("s")
    pltpu.sync_copy(ss_h, ssbuf)
    _zero_vmem_2d(zbuf, 64)
    scale = [ssbuf[pl.ds(g * 16, 16)] for g in range(8)]
    shift = [ssbuf[pl.ds(128 + g * 16, 16)] for g in range(8)]
    base = pl.multiple_of(s * PRT, 8)
    for k in range(5):
        pltpu.sync_copy(zbuf, acc_sh.at[pl.ds(base + k * 64, 64)])
    plsc.subcore_barrier()

    for j in range(16):
        cid = NCH5C * c + s + NS * j

        @pl.when(cid < NCH5C * (c + 1))
        def _():
            a = pl.multiple_of(cid * CE, 8)
            pltpu.sync_copy(dm1_h.at[pl.ds(a, CE)], i1)
            pltpu.sync_copy(m_h.at[pl.ds(a, CE)], mbuf)

            def row(r, _):
                for g in range(8):
                    col = pl.ds(g * 16, 16)
                    v = mbuf[r, col]
                    mbuf[r, col] = jnp.maximum(v * scale[g] + shift[g], 0.0)
                return 0

            lax.fori_loop(0, CE, row, 0, unroll=False)
            pltpu.async_copy(mbuf, acc_sh.at[i1], sem, add=True).wait()

    plsc.subcore_barrier()
    pltpu.sync_copy(acc_sh.at[pl.ds(base, PRT)],
                    pres_h.at[c, pl.ds(base, PRT)])


_k5 = pl.kernel(
    _k5_body,
    out_type=jax.ShapeDtypeStruct((NC, PR, H), _f32),
    mesh=_mesh,
    scratch_types=[
        pltpu.VMEM((CE,), _i32),
        pltpu.VMEM((CE, H), _f32),
        pltpu.VMEM((64, H), _f32),
        pltpu.VMEM((2 * H,), _f32),
        pltpu.VMEM_SHARED((PR, H), _f32),
        pltpu.SemaphoreType.DMA,
    ],
)


def _k6_body(p_ref, o_ref):
    o_ref[...] = p_ref[0, :NY, :] + p_ref[1, :NY, :]


def _k6(pres):
    return pl.pallas_call(
        _k6_body,
        out_shape=jax.ShapeDtypeStruct((NY, H), _f32),
    )(pres)


def kernel(x, y, source_domain_indicator, node_map_edge_index,
           intersect_indicator, domain_map_edge_index, W_xsum, W_xint, W_y,
           bn_weight, bn_bias):
    nm0 = node_map_edge_index[0].astype(_i32)
    dm0 = domain_map_edge_index[0].astype(_i32)
    dm1 = domain_map_edge_index[1].astype(_i32)
    ii = intersect_indicator.astype(_i32)
    sdi = source_domain_indicator.astype(_i32)

    e_src = jnp.concatenate(
        [jnp.arange(NX, dtype=_i32), nm0, jnp.zeros((EPAD,), _i32)])
    e_seg = jnp.concatenate([sdi, ii + NSRC_PAD])
    e_loc = jnp.concatenate([e_seg & (SB - 1), jnp.full((EPAD,), SB, _i32)])
    bounds = jnp.searchsorted(
        e_seg, jnp.arange(NB + 1, dtype=_i32) * SB).astype(_i32)
    bounds = jnp.concatenate([bounds, jnp.full((16,), ENX, _i32)])

    sums = _k1(x, e_src, e_loc, bounds)
    b_mat, a_mat, c_mat = _k2(sums, y, W_xint, W_xsum, W_y)
    m, parts = _k3(b_mat, a_mat, c_mat, dm0, dm1)
    ss = _k4(parts.reshape(NW, 2 * H), bn_weight.reshape(1, H),
             bn_bias.reshape(1, H))
    pres = _k5(m, dm1, ss.reshape(2 * H))
    return _k6(pres)


# speedup vs baseline: 7.5810x; 1.2880x over previous
import functools

import jax
import jax.numpy as jnp
from jax import lax
from jax.experimental import pallas as pl
from jax.experimental.pallas import tpu as pltpu
from jax.experimental.pallas import tpu_sc as plsc

H = 128
NX = 10000
NY = 5000
NSRC = 5000
EN = 640000
ED = 64000

NC = 2
NS = 16
NW = NC * NS

SB = 256
NSRC_PAD = 5120
NSEG = NSRC_PAD + ED
NB = NSEG // SB
ENX = EN + NX
CE = 128
NLIVE = 4
WIN = 32
EPAD = WIN * CE + 256
ACC_ROWS = SB + 8

PR = 5120
PRT = PR // NS

_mesh = plsc.VectorSubcoreMesh(core_axis_name="c", subcore_axis_name="s")
_f32 = jnp.float32
_i32 = jnp.int32


def _zero_vmem_2d(ref, rows):
    def body(i, _):
        r = i // (H // 16)
        g = i % (H // 16)
        ref[r, pl.ds(g * 16, 16)] = jnp.zeros((16,), _f32)
        return 0
    lax.fori_loop(0, rows * (H // 16), body, 0, unroll=False)


def _k1_body(x_h, src_h, loc_h, bnd_h, out_h,
             bnd_v, sbufw, lbufw, lbuf2, rowbuf, zbuf, acc_sh,
             gsems, ssems):
    c = lax.axis_index("c")
    s = lax.axis_index("s")
    wid = s * NC + c
    pltpu.sync_copy(bnd_h, bnd_v)
    _zero_vmem_2d(zbuf, 88)
    iota = lax.broadcasted_iota(_i32, (16,), 0)
    tile_base = pl.multiple_of(s * ACC_ROWS, 8)

    for bi in range(-(-NB // NW)):
        b = wid + NW * bi

        @pl.when(b < NB)
        def _():
            bv = bnd_v[pl.ds(b, 16)]
            lo = bv[0]
            hi = bv[1]
            for r in range(3):
                pltpu.sync_copy(
                    zbuf, acc_sh.at[pl.ds(tile_base + r * 88, 88)])
            a0 = pl.multiple_of(lo & (-CE), 8)
            nch = (hi - a0 + CE - 1) // CE
            nwin = (nch + WIN - 1) // WIN

            def win(w, _):
                aw = pl.multiple_of(a0 + w * (WIN * CE), 8)
                pltpu.sync_copy(src_h.at[pl.ds(aw, WIN * CE)], sbufw)
                pltpu.sync_copy(loc_h.at[pl.ds(aw, WIN * CE)], lbufw)
                nrem = nch - w * WIN
                ngrp = lax.min((nrem + NLIVE - 1) // NLIVE,
                               jnp.int32(WIN // NLIVE))

                def grp(g, _):
                    q0 = g * NLIVE
                    for k in range(NLIVE):
                        @pl.when(q0 + k < nrem)
                        def _():
                            pltpu.async_copy(
                                x_h.at[sbufw.at[pl.ds(pl.multiple_of((q0 + k) * CE, 8), CE)]],
                                rowbuf.at[k], gsems[k])
                    for k in range(NLIVE):
                        @pl.when(q0 + k < nrem)
                        def _():
                            pltpu.make_async_copy(
                                x_h.at[sbufw.at[pl.ds(pl.multiple_of((q0 + k) * CE, 8), CE)]],
                                rowbuf.at[k], gsems[k]).wait()
                            a = aw + (q0 + k) * CE
                            for g8 in range(CE // 16):
                                e = a + g8 * 16 + iota
                                lv = lbufw[pl.ds((q0 + k) * CE + g8 * 16, 16)]
                                ok = (e >= lo) & (e < hi)
                                lbuf2[k, pl.ds(g8 * 16, 16)] = (
                                    tile_base + jnp.where(ok, lv, SB))
                            pltpu.async_copy(rowbuf.at[k],
                                             acc_sh.at[lbuf2.at[k]],
                                             ssems[k], add=True)
                    for k in range(NLIVE):
                        @pl.when(q0 + k < nrem)
                        def _():
                            pltpu.make_async_copy(
                                rowbuf.at[k], acc_sh.at[lbuf2.at[k]],
                                ssems[k]).wait()
                    return 0

                lax.fori_loop(0, ngrp, grp, 0, unroll=False)
                return 0

            lax.fori_loop(0, nwin, win, 0, unroll=False)
            pltpu.sync_copy(acc_sh.at[pl.ds(tile_base, SB)],
                            out_h.at[pl.ds(pl.multiple_of(b * SB, 8), SB)])


_k1 = pl.kernel(
    _k1_body,
    out_type=jax.ShapeDtypeStruct((NSEG, H), _f32),
    mesh=_mesh,
    scratch_types=[
        pltpu.VMEM((NB + 17,), _i32),
        pltpu.VMEM((WIN * CE,), _i32),
        pltpu.VMEM((WIN * CE,), _i32),
        pltpu.VMEM((NLIVE, CE), _i32),
        pltpu.VMEM((NLIVE, CE, H), _f32),
        pltpu.VMEM((88, H), _f32),
        pltpu.VMEM_SHARED((NS * ACC_ROWS, H), _f32),
        [pltpu.SemaphoreType.DMA] * NLIVE,
        [pltpu.SemaphoreType.DMA] * NLIVE,
    ],
)


def _k2_body(xi_ref, sa_ref, y_ref, wi_ref, ws_ref, wy_ref,
             b_out, a_out, c_out):
    i = pl.program_id(0)
    b_out[...] = jnp.dot(xi_ref[...], wi_ref[...],
                         precision=lax.Precision.HIGHEST)

    @pl.when(i == 0)
    def _():
        a_out[...] = jnp.dot(sa_ref[...], ws_ref[...],
                             precision=lax.Precision.HIGHEST)
        wy = wy_ref[:, :H] + wy_ref[:, H:]
        c_out[...] = jnp.dot(y_ref[...], wy, precision=lax.Precision.HIGHEST)


def _k2(sums, y, w_xint, w_xsum, w_y):
    nblk = ED // SB
    return pl.pallas_call(
        _k2_body,
        grid=(nblk,),
        in_specs=[
            pl.BlockSpec((SB, H), lambda i: (NSRC_PAD // SB + i, 0)),
            pl.BlockSpec((NSRC_PAD, H), lambda i: (0, 0)),
            pl.BlockSpec((NY, H), lambda i: (0, 0)),
            pl.BlockSpec((H, H), lambda i: (0, 0)),
            pl.BlockSpec((H, H), lambda i: (0, 0)),
            pl.BlockSpec((H, 2 * H), lambda i: (0, 0)),
        ],
        out_specs=[
            pl.BlockSpec((SB, H), lambda i: (i, 0)),
            pl.BlockSpec((NSRC_PAD, H), lambda i: (0, 0)),
            pl.BlockSpec((NY, H), lambda i: (0, 0)),
        ],
        out_shape=[
            jax.ShapeDtypeStruct((ED, H), _f32),
            jax.ShapeDtypeStruct((NSRC_PAD, H), _f32),
            jax.ShapeDtypeStruct((NY, H), _f32),
        ],
    )(sums, sums, y, w_xint, w_xsum, w_y)


NCH3 = ED // CE


def _k3_body(b_h, a_h, c_h, dm0_h, dm1_h, m_h, part_h,
             i0, i1, abuf, cbuf, bbuf, pbuf, sa, sc, sb, sm):
    c = lax.axis_index("c")
    s = lax.axis_index("s")
    wid = s * NC + c
    for g in range(16):
        pbuf[pl.ds(g * 16, 16)] = jnp.zeros((16,), _f32)

    nj = -(-NCH3 // NW)

    def _cid(j):
        return wid + NW * j

    def _stage(j):
        p = j & 1

        @pl.when(_cid(j) < NCH3)
        def _():
            a = pl.multiple_of(_cid(j) * CE, 8)
            pltpu.sync_copy(dm0_h.at[pl.ds(a, CE)], i0.at[p])
            pltpu.sync_copy(dm1_h.at[pl.ds(a, CE)], i1.at[p])
            pltpu.async_copy(a_h.at[i0.at[p]], abuf.at[p], sa[p])
            pltpu.async_copy(c_h.at[i1.at[p]], cbuf.at[p], sc[p])
            pltpu.async_copy(b_h.at[pl.ds(a, CE)], bbuf.at[p], sb[p])

    _stage(0)
    for j in range(nj):
        p = j & 1
        if j > 0:
            @pl.when(_cid(j - 1) < NCH3)
            def _():
                ap = pl.multiple_of(_cid(j - 1) * CE, 8)
                pltpu.make_async_copy(
                    bbuf.at[1 - p], m_h.at[pl.ds(ap, CE)], sm[1 - p]).wait()
        if j + 1 < nj:
            _stage(j + 1)

        @pl.when(_cid(j) < NCH3)
        def _():
            a = pl.multiple_of(_cid(j) * CE, 8)
            pltpu.make_async_copy(a_h.at[i0.at[p]], abuf.at[p], sa[p]).wait()
            pltpu.make_async_copy(c_h.at[i1.at[p]], cbuf.at[p], sc[p]).wait()
            pltpu.make_async_copy(b_h.at[pl.ds(a, CE)], bbuf.at[p],
                                  sb[p]).wait()

            def row(r, carry):
                out = []
                for g in range(8):
                    col = pl.ds(g * 16, 16)
                    v = (bbuf[p, r, col] + abuf[p, r, col]
                         + cbuf[p, r, col])
                    bbuf[p, r, col] = v
                    out.append(carry[g] + v)
                    out.append(carry[8 + g] + v * v)
                return tuple(out[0::2]) + tuple(out[1::2])

            z16 = tuple(jnp.zeros((16,), _f32) for _ in range(16))
            acc = lax.fori_loop(0, CE, row, z16, unroll=False)
            for g in range(8):
                pb = pbuf[pl.ds(g * 16, 16)]
                pbuf[pl.ds(g * 16, 16)] = pb + acc[g]
                q = pbuf[pl.ds(128 + g * 16, 16)]
                pbuf[pl.ds(128 + g * 16, 16)] = q + acc[8 + g]
            pltpu.async_copy(bbuf.at[p], m_h.at[pl.ds(a, CE)], sm[p])

    jl = nj - 1
    pz = jl & 1

    @pl.when(_cid(jl) < NCH3)
    def _():
        a = pl.multiple_of(_cid(jl) * CE, 8)
        pltpu.make_async_copy(bbuf.at[pz], m_h.at[pl.ds(a, CE)],
                              sm[pz]).wait()

    pltpu.sync_copy(pbuf,
                    part_h.at[pl.ds(pl.multiple_of(wid * 2 * H, 8), 2 * H)])


_k3 = pl.kernel(
    _k3_body,
    out_type=(
        jax.ShapeDtypeStruct((ED, H), _f32),
        jax.ShapeDtypeStruct((NW * 2 * H,), _f32),
    ),
    mesh=_mesh,
    scratch_types=[
        pltpu.VMEM((2, CE), _i32),
        pltpu.VMEM((2, CE), _i32),
        pltpu.VMEM((2, CE, H), _f32),
        pltpu.VMEM((2, CE, H), _f32),
        pltpu.VMEM((2, CE, H), _f32),
        pltpu.VMEM((2 * H,), _f32),
        [pltpu.SemaphoreType.DMA] * 2,
        [pltpu.SemaphoreType.DMA] * 2,
        [pltpu.SemaphoreType.DMA] * 2,
        [pltpu.SemaphoreType.DMA] * 2,
    ],
)


def _k4_body(p_ref, w_ref, b_ref, o_ref):
    p = p_ref[...]
    tot = jnp.sum(p, axis=0, keepdims=True)
    s1 = tot[:, :H]
    s2 = tot[:, H:]
    mean = s1 / float(ED)
    var = s2 / float(ED) - mean * mean
    inv = lax.rsqrt(var + 1e-5)
    scale = w_ref[...] * inv
    shift = b_ref[...] - mean * scale
    o_ref[0:1, :] = scale
    o_ref[1:2, :] = shift


def _k4(parts, bnw, bnb):
    return pl.pallas_call(
        _k4_body,
        out_shape=jax.ShapeDtypeStruct((2, H), _f32),
    )(parts, bnw, bnb)


NCH5 = ED // CE
NCH5C = NCH5 // NC


def _k5_body(m_h, dm1_h, ss_h, pres_h,
             i1, mbuf, zbuf, ssbuf, acc_sh, sem, sl):
    c = lax.axis_index("c")
    s = lax.axis_index("s")
    pltpu.sync_copy(ss_h, ssbuf)
    _zero_vmem_2d(zbuf, 64)
    scale = [ssbuf[pl.ds(g * 16, 16)] for g in range(8)]
    shift = [ssbuf[pl.ds(128 + g * 16, 16)] for g in range(8)]
    base = pl.multiple_of(s * PRT, 8)
    for k in range(5):
        pltpu.sync_copy(zbuf, acc_sh.at[pl.ds(base + k * 64, 64)])
    plsc.subcore_barrier()

    nj = 16

    def _cid(j):
        return NCH5C * c + s + NS * j

    def _valid(j):
        return _cid(j) < NCH5C * (c + 1)

    def _stage(j):
        p = j & 1

        @pl.when(_valid(j))
        def _():
            a = pl.multiple_of(_cid(j) * CE, 8)
            pltpu.sync_copy(dm1_h.at[pl.ds(a, CE)], i1.at[p])
            pltpu.async_copy(m_h.at[pl.ds(a, CE)], mbuf.at[p], sl[p])

    _stage(0)
    for j in range(nj):
        p = j & 1
        if j > 0:
            @pl.when(_valid(j - 1))
            def _():
                pltpu.make_async_copy(mbuf.at[1 - p],
                                      acc_sh.at[i1.at[1 - p]],
                                      sem[1 - p]).wait()
        if j + 1 < nj:
            _stage(j + 1)

        @pl.when(_valid(j))
        def _():
            a = pl.multiple_of(_cid(j) * CE, 8)
            pltpu.make_async_copy(m_h.at[pl.ds(a, CE)], mbuf.at[p],
                                  sl[p]).wait()

            def row(r, _):
                for g in range(8):
                    col = pl.ds(g * 16, 16)
                    v = mbuf[p, r, col]
                    mbuf[p, r, col] = jnp.maximum(
                        v * scale[g] + shift[g], 0.0)
                return 0

            lax.fori_loop(0, CE, row, 0, unroll=False)
            pltpu.async_copy(mbuf.at[p], acc_sh.at[i1.at[p]], sem[p],
                             add=True)

    jl = nj - 1
    pz = jl & 1

    @pl.when(_valid(jl))
    def _():
        pltpu.make_async_copy(mbuf.at[pz], acc_sh.at[i1.at[pz]],
                              sem[pz]).wait()

    plsc.subcore_barrier()
    pltpu.sync_copy(acc_sh.at[pl.ds(base, PRT)],
                    pres_h.at[c, pl.ds(base, PRT)])


_k5 = pl.kernel(
    _k5_body,
    out_type=jax.ShapeDtypeStruct((NC, PR, H), _f32),
    mesh=_mesh,
    scratch_types=[
        pltpu.VMEM((2, CE), _i32),
        pltpu.VMEM((2, CE, H), _f32),
        pltpu.VMEM((64, H), _f32),
        pltpu.VMEM((2 * H,), _f32),
        pltpu.VMEM_SHARED((PR, H), _f32),
        [pltpu.SemaphoreType.DMA] * 2,
        [pltpu.SemaphoreType.DMA] * 2,
    ],
)


def _k6_body(p_ref, o_ref):
    o_ref[...] = p_ref[0, :NY, :] + p_ref[1, :NY, :]


def _k6(pres):
    return pl.pallas_call(
        _k6_body,
        out_shape=jax.ShapeDtypeStruct((NY, H), _f32),
    )(pres)


def kernel(x, y, source_domain_indicator, node_map_edge_index,
           intersect_indicator, domain_map_edge_index, W_xsum, W_xint, W_y,
           bn_weight, bn_bias):
    nm0 = node_map_edge_index[0].astype(_i32)
    dm0 = domain_map_edge_index[0].astype(_i32)
    dm1 = domain_map_edge_index[1].astype(_i32)
    ii = intersect_indicator.astype(_i32)
    sdi = source_domain_indicator.astype(_i32)

    e_src = jnp.concatenate(
        [jnp.arange(NX, dtype=_i32), nm0, jnp.zeros((EPAD,), _i32)])
    e_seg = jnp.concatenate([sdi, ii + NSRC_PAD])
    e_loc = jnp.concatenate([e_seg & (SB - 1), jnp.full((EPAD,), SB, _i32)])
    bounds = jnp.searchsorted(
        e_seg, jnp.arange(NB + 1, dtype=_i32) * SB).astype(_i32)
    bounds = jnp.concatenate([bounds, jnp.full((16,), ENX, _i32)])

    sums = _k1(x, e_src, e_loc, bounds)
    b_mat, a_mat, c_mat = _k2(sums, y, W_xint, W_xsum, W_y)
    m, parts = _k3(b_mat, a_mat, c_mat, dm0, dm1)
    ss = _k4(parts.reshape(NW, 2 * H), bn_weight.reshape(1, H),
             bn_bias.reshape(1, H))
    pres = _k5(m, dm1, ss.reshape(2 * H))
    return _k6(pres)
